# baseline (speedup 1.0000x reference)
import functools

import jax
import jax.numpy as jnp
from jax import lax
from jax.experimental import pallas as pl
from jax.experimental.pallas import tpu as pltpu
from jax.experimental.pallas import tpu_sc as plsc

B = 1024
L = 200
V = 100000
D = 64

NC = 2
NS = 16
NW = NC * NS
LH = L // 2
BPW = B // NW
NCH = BPW
C = LH


def _sc_body(tok_hbm, civ_hbm, ttab_hbm, comb_hbm, out_hbm,
             idx_v, civ_v, comb_sh, rows0, rows1, add0, add1, ob0, ob1,
             gsr0, gsa0, gsr1, gsa1, os0, os1):
  wid = lax.axis_index("c") * NS + lax.axis_index("s")

  @pl.when(lax.axis_index("s") == 0)
  def _():
    pltpu.sync_copy(comb_hbm, comb_sh)
  plsc.subcore_barrier()

  pltpu.sync_copy(tok_hbm.at[wid], idx_v)
  pltpu.sync_copy(civ_hbm.at[wid], civ_v)

  def start_gathers(c, rows, add, gsr, gsa):
    pltpu.async_copy(ttab_hbm.at[idx_v.at[c]], rows, gsr)
    pltpu.async_copy(comb_sh.at[civ_v.at[c]], add, gsa)

  def wait_gathers(rows, add, gsr, gsa):
    pltpu.make_async_copy(ttab_hbm.at[idx_v.at[0]], rows, gsr).wait()
    pltpu.make_async_copy(comb_sh.at[civ_v.at[0]], add, gsa).wait()

  def compute(rows, add, ob):
    def pairrow(p, rc):
      r0 = 2 * p
      r1 = r0 + 1
      sls = [pl.ds(j * 16, 16) for j in range(D // 16)]
      a = [rows[r0, sl] for sl in sls] + [rows[r1, sl] for sl in sls]
      b = [add[r0, sl] for sl in sls] + [add[r1, sl] for sl in sls]
      s = [x + y for x, y in zip(a, b)]
      for j in range(D // 16):
        ob[r0, sls[j]] = s[j]
        ob[r1, sls[j]] = s[D // 16 + j]
      return rc
    lax.fori_loop(0, C // 2, pairrow, 0)

  def slot(h, c, rows, add, ob, gsr, gsa, osem):
    wait_gathers(rows, add, gsr, gsa)

    @pl.when(h > 0)
    def _():
      pltpu.make_async_copy(ob, out_hbm.at[0], osem).wait()

    compute(rows, add, ob)

    @pl.when(h < NCH // 2 - 1)
    def _():
      start_gathers(c + 2, rows, add, gsr, gsa)

    pltpu.async_copy(ob, out_hbm.at[wid * NCH + c], osem)

  start_gathers(0, rows0, add0, gsr0, gsa0)
  start_gathers(1, rows1, add1, gsr1, gsa1)

  def pair(h, carry):
    slot(h, 2 * h, rows0, add0, ob0, gsr0, gsa0, os0)
    slot(h, 2 * h + 1, rows1, add1, ob1, gsr1, gsa1, os1)
    return carry
  lax.fori_loop(0, NCH // 2, pair, 0)

  pltpu.make_async_copy(ob0, out_hbm.at[0], os0).wait()
  pltpu.make_async_copy(ob1, out_hbm.at[0], os1).wait()


@jax.jit
def _sc_embed_half(tok3, civ3, ttab, comb):
  return pl.kernel(
      _sc_body,
      out_type=jax.ShapeDtypeStruct((B, LH, D), jnp.float32),
      mesh=plsc.VectorSubcoreMesh(core_axis_name="c", subcore_axis_name="s"),
      compiler_params=pltpu.CompilerParams(use_tc_tiling_on_sc=False),
      scratch_types=[
          pltpu.VMEM((NCH, C), jnp.int32),
          pltpu.VMEM((NCH, C), jnp.int32),
          pltpu.VMEM_SHARED((2 * L, D), jnp.float32),
          pltpu.VMEM((C, D), jnp.float32),
          pltpu.VMEM((C, D), jnp.float32),
          pltpu.VMEM((C, D), jnp.float32),
          pltpu.VMEM((C, D), jnp.float32),
          pltpu.VMEM((C, D), jnp.float32),
          pltpu.VMEM((C, D), jnp.float32),
          pltpu.SemaphoreType.DMA,
          pltpu.SemaphoreType.DMA,
          pltpu.SemaphoreType.DMA,
          pltpu.SemaphoreType.DMA,
          pltpu.SemaphoreType.DMA,
          pltpu.SemaphoreType.DMA,
      ],
  )(tok3, civ3, ttab, comb)


def _prep_body(ids_ref, typ_ref, seg_ref, pos_ref, mask_ref, comb_ref, civ_ref):
  mask_ref[...] = ids_ref[...] != 0
  l_iota = lax.broadcasted_iota(jnp.int32, (B, L), 1)
  civ_ref[...] = typ_ref[...] * L + l_iota
  seg = seg_ref[...]
  pos = pos_ref[...]
  comb_ref[...] = jnp.concatenate([pos + seg[0:1, :], pos + seg[1:2, :]],
                                  axis=0)


@jax.jit
def _prep_call(token_ids, type_token_ids, segment_table, positional_table):
  return pl.pallas_call(
      _prep_body,
      out_shape=(
          jax.ShapeDtypeStruct((B, L), jnp.bool_),
          jax.ShapeDtypeStruct((2 * L, D), jnp.float32),
          jax.ShapeDtypeStruct((B, L), jnp.int32),
      ),
  )(token_ids, type_token_ids, segment_table, positional_table)


def kernel(token_ids, type_token_ids, token_table, segment_table, positional_table):
  token_ids = token_ids.astype(jnp.int32)
  type_token_ids = type_token_ids.astype(jnp.int32)
  mask, comb, civ = _prep_call(token_ids, type_token_ids, segment_table,
                               positional_table)
  halves = []
  for h in range(2):
    lsl = slice(h * LH, (h + 1) * LH)
    tok3 = token_ids[:, lsl].reshape(NW, NCH, C)
    civ3 = civ[:, lsl].reshape(NW, NCH, C)
    halves.append(_sc_embed_half(tok3, civ3, token_table, comb))
  outputs = jnp.concatenate(halves, axis=1)
  attention_mask = mask.reshape(B, 1, 1, L)
  return outputs, attention_mask


# speedup vs baseline: 1.2700x; 1.2700x over previous
import functools

import jax
import jax.numpy as jnp
from jax import lax
from jax.experimental import pallas as pl
from jax.experimental.pallas import tpu as pltpu
from jax.experimental.pallas import tpu_sc as plsc

B = 1024
L = 200
V = 100000
D = 64

NC = 2
NS = 16
NW = NC * NS
NBL = B * L
PW = NBL // NW
C = 200
NCH = PW // C
CH = C // 2

def _sc_body(tok_hbm, civ_hbm, ttab_hbm, comb_hbm, out_hbm,
             idx_v, civ_v, comb_sh, rows0, rows1, add0, add1, ob0, ob1,
             gsr0, gsa0, gsr1, gsa1, os0, os1):
  wid = lax.axis_index("c") * NS + lax.axis_index("s")

  @pl.when(lax.axis_index("s") == 0)
  def _():
    pltpu.sync_copy(comb_hbm, comb_sh)
  plsc.subcore_barrier()

  pltpu.sync_copy(tok_hbm.at[wid], idx_v)
  pltpu.sync_copy(civ_hbm.at[wid], civ_v)

  def start_gathers(c, rows, add, gsr, gsa):
    pltpu.async_copy(ttab_hbm.at[idx_v.at[c, 0]], rows.at[pl.ds(0, CH)], gsr)
    pltpu.async_copy(ttab_hbm.at[idx_v.at[c, 1]], rows.at[pl.ds(CH, CH)], gsr)
    pltpu.async_copy(comb_sh.at[civ_v.at[c, 0]], add.at[pl.ds(0, CH)], gsa)
    pltpu.async_copy(comb_sh.at[civ_v.at[c, 1]], add.at[pl.ds(CH, CH)], gsa)

  def wait_gathers(rows, add, gsr, gsa):
    for k in range(2):
      sl = pl.ds(k * CH, CH)
      pltpu.make_async_copy(ttab_hbm.at[idx_v.at[0, 0]], rows.at[sl],
                            gsr).wait()
      pltpu.make_async_copy(comb_sh.at[civ_v.at[0, 0]], add.at[sl],
                            gsa).wait()

  def compute(rows, add, ob):
    def pairrow(p, rc):
      r0 = 2 * p
      r1 = r0 + 1
      sls = [pl.ds(j * 16, 16) for j in range(D // 16)]
      a = [rows[r0, sl] for sl in sls] + [rows[r1, sl] for sl in sls]
      b = [add[r0, sl] for sl in sls] + [add[r1, sl] for sl in sls]
      s = [x + y for x, y in zip(a, b)]
      for j in range(D // 16):
        ob[r0, sls[j]] = s[j]
        ob[r1, sls[j]] = s[D // 16 + j]
      return rc
    lax.fori_loop(0, C // 2, pairrow, 0)

  def slot(h, c, rows, add, ob, gsr, gsa, osem):
    wait_gathers(rows, add, gsr, gsa)

    @pl.when(h > 0)
    def _():
      pltpu.make_async_copy(ob, out_hbm.at[0], osem).wait()

    compute(rows, add, ob)

    @pl.when(h < NCH // 2 - 1)
    def _():
      start_gathers(c + 2, rows, add, gsr, gsa)

    pltpu.async_copy(ob, out_hbm.at[wid * NCH + c], osem)

  start_gathers(0, rows0, add0, gsr0, gsa0)
  start_gathers(1, rows1, add1, gsr1, gsa1)

  def pair(h, carry):
    slot(h, 2 * h, rows0, add0, ob0, gsr0, gsa0, os0)
    slot(h, 2 * h + 1, rows1, add1, ob1, gsr1, gsa1, os1)
    return carry
  lax.fori_loop(0, NCH // 2, pair, 0)

  pltpu.make_async_copy(ob0, out_hbm.at[0], os0).wait()
  pltpu.make_async_copy(ob1, out_hbm.at[0], os1).wait()


@jax.jit
def _sc_embed(tok4, civ4, ttab, comb):
  return pl.kernel(
      _sc_body,
      out_type=jax.ShapeDtypeStruct((B, L, D), jnp.float32),
      mesh=plsc.VectorSubcoreMesh(core_axis_name="c", subcore_axis_name="s"),
      compiler_params=pltpu.CompilerParams(use_tc_tiling_on_sc=False),
      scratch_types=[
          pltpu.VMEM((NCH, 2, CH), jnp.int32),
          pltpu.VMEM((NCH, 2, CH), jnp.int32),
          pltpu.VMEM_SHARED((2 * L, D), jnp.float32),
          pltpu.VMEM((C, D), jnp.float32),
          pltpu.VMEM((C, D), jnp.float32),
          pltpu.VMEM((C, D), jnp.float32),
          pltpu.VMEM((C, D), jnp.float32),
          pltpu.VMEM((C, D), jnp.float32),
          pltpu.VMEM((C, D), jnp.float32),
          pltpu.SemaphoreType.DMA,
          pltpu.SemaphoreType.DMA,
          pltpu.SemaphoreType.DMA,
          pltpu.SemaphoreType.DMA,
          pltpu.SemaphoreType.DMA,
          pltpu.SemaphoreType.DMA,
      ],
  )(tok4, civ4, ttab, comb)


def _prep_body(ids_ref, seg_ref, pos_ref, mask_ref, comb_ref):
  mask_ref[...] = ids_ref[...] != 0
  seg = seg_ref[...]
  pos = pos_ref[...]
  comb_ref[...] = jnp.concatenate([pos + seg[0:1, :], pos + seg[1:2, :]],
                                  axis=0)


@jax.jit
def _prep_call(token_ids, segment_table, positional_table):
  return pl.pallas_call(
      _prep_body,
      out_shape=(
          jax.ShapeDtypeStruct((B, L), jnp.bool_),
          jax.ShapeDtypeStruct((2 * L, D), jnp.float32),
      ),
  )(token_ids, segment_table, positional_table)


def kernel(token_ids, type_token_ids, token_table, segment_table, positional_table):
  token_ids = token_ids.astype(jnp.int32)
  type_token_ids = type_token_ids.astype(jnp.int32)
  mask, comb = _prep_call(token_ids, segment_table, positional_table)
  civ = type_token_ids * L + jnp.arange(L, dtype=jnp.int32)[None, :]
  tok4 = token_ids.reshape(NW, NCH, 2, CH)
  civ4 = civ.reshape(NW, NCH, 2, CH)
  outputs = _sc_embed(tok4, civ4, token_table, comb)
  attention_mask = mask.reshape(B, 1, 1, L)
  return outputs, attention_mask


# speedup vs baseline: 1.2707x; 1.0006x over previous
import jax
import jax.numpy as jnp
from jax import lax
from jax.experimental import pallas as pl
from jax.experimental.pallas import tpu as pltpu
from jax.experimental.pallas import tpu_sc as plsc

B = 1024
L = 200
V = 100000
D = 64

NC = 2
NS = 16
NW = NC * NS
NBL = B * L
PW = NBL // NW
C = 200
NCH = PW // C
CH = C // 2

def _sc_body(tok_hbm, civ_hbm, ttab_hbm, comb_hbm, out_hbm,
             idx_v, civ_v, comb_sh, rows0, rows1, add0, add1, ob0, ob1,
             gsr0, gsa0, gsr1, gsa1, os0, os1):
  wid = lax.axis_index("c") * NS + lax.axis_index("s")

  @pl.when(lax.axis_index("s") == 0)
  def _():
    pltpu.sync_copy(comb_hbm, comb_sh)
  plsc.subcore_barrier()

  pltpu.sync_copy(tok_hbm.at[wid], idx_v)
  pltpu.sync_copy(civ_hbm.at[wid], civ_v)

  def start_gathers(c, rows, add, gsr, gsa):
    pltpu.async_copy(ttab_hbm.at[idx_v.at[c, 0]], rows.at[pl.ds(0, CH)], gsr)
    pltpu.async_copy(ttab_hbm.at[idx_v.at[c, 1]], rows.at[pl.ds(CH, CH)], gsr)
    pltpu.async_copy(comb_sh.at[civ_v.at[c, 0]], add.at[pl.ds(0, CH)], gsa)
    pltpu.async_copy(comb_sh.at[civ_v.at[c, 1]], add.at[pl.ds(CH, CH)], gsa)

  def wait_gathers(rows, add, gsr, gsa):
    for k in range(2):
      sl = pl.ds(k * CH, CH)
      pltpu.make_async_copy(ttab_hbm.at[idx_v.at[0, 0]], rows.at[sl],
                            gsr).wait()
      pltpu.make_async_copy(comb_sh.at[civ_v.at[0, 0]], add.at[sl],
                            gsa).wait()

  def compute(rows, add, ob):
    def pairrow(p, rc):
      r0 = 2 * p
      r1 = r0 + 1
      sls = [pl.ds(j * 16, 16) for j in range(D // 16)]
      a = [rows[r0, sl] for sl in sls] + [rows[r1, sl] for sl in sls]
      b = [add[r0, sl] for sl in sls] + [add[r1, sl] for sl in sls]
      s = [x + y for x, y in zip(a, b)]
      for j in range(D // 16):
        ob[r0, sls[j]] = s[j]
        ob[r1, sls[j]] = s[D // 16 + j]
      return rc
    lax.fori_loop(0, C // 2, pairrow, 0)

  def slot(h, c, rows, add, ob, gsr, gsa, osem):
    wait_gathers(rows, add, gsr, gsa)

    @pl.when(h > 0)
    def _():
      pltpu.make_async_copy(ob, out_hbm.at[0], osem).wait()

    compute(rows, add, ob)

    @pl.when(h < NCH // 2 - 1)
    def _():
      start_gathers(c + 2, rows, add, gsr, gsa)

    pltpu.async_copy(ob, out_hbm.at[wid * NCH + c], osem)

  start_gathers(0, rows0, add0, gsr0, gsa0)
  start_gathers(1, rows1, add1, gsr1, gsa1)

  def pair(h, carry):
    slot(h, 2 * h, rows0, add0, ob0, gsr0, gsa0, os0)
    slot(h, 2 * h + 1, rows1, add1, ob1, gsr1, gsa1, os1)
    return carry
  lax.fori_loop(0, NCH // 2, pair, 0)

  pltpu.make_async_copy(ob0, out_hbm.at[0], os0).wait()
  pltpu.make_async_copy(ob1, out_hbm.at[0], os1).wait()


@jax.jit
def _sc_embed(tok4, civ4, ttab, comb):
  return pl.kernel(
      _sc_body,
      out_type=jax.ShapeDtypeStruct((B, L, D), jnp.float32),
      mesh=plsc.VectorSubcoreMesh(core_axis_name="c", subcore_axis_name="s"),
      compiler_params=pltpu.CompilerParams(use_tc_tiling_on_sc=False),
      scratch_types=[
          pltpu.VMEM((NCH, 2, CH), jnp.int32),
          pltpu.VMEM((NCH, 2, CH), jnp.int32),
          pltpu.VMEM_SHARED((2 * L, D), jnp.float32),
          pltpu.VMEM((C, D), jnp.float32),
          pltpu.VMEM((C, D), jnp.float32),
          pltpu.VMEM((C, D), jnp.float32),
          pltpu.VMEM((C, D), jnp.float32),
          pltpu.VMEM((C, D), jnp.float32),
          pltpu.VMEM((C, D), jnp.float32),
          pltpu.SemaphoreType.DMA,
          pltpu.SemaphoreType.DMA,
          pltpu.SemaphoreType.DMA,
          pltpu.SemaphoreType.DMA,
          pltpu.SemaphoreType.DMA,
          pltpu.SemaphoreType.DMA,
      ],
  )(tok4, civ4, ttab, comb)


def _prep_body(ids_ref, seg_ref, pos_ref, mask_ref, comb_ref):
  mask_ref[...] = ids_ref[...] != 0
  seg = seg_ref[...]
  pos = pos_ref[...]
  comb_ref[...] = jnp.concatenate([pos + seg[0:1, :], pos + seg[1:2, :]],
                                  axis=0)


@jax.jit
def _prep_call(token_ids, segment_table, positional_table):
  return pl.pallas_call(
      _prep_body,
      out_shape=(
          jax.ShapeDtypeStruct((B, L), jnp.bool_),
          jax.ShapeDtypeStruct((2 * L, D), jnp.float32),
      ),
  )(token_ids, segment_table, positional_table)


def kernel(token_ids, type_token_ids, token_table, segment_table, positional_table):
  token_ids = token_ids.astype(jnp.int32)
  type_token_ids = type_token_ids.astype(jnp.int32)
  mask, comb = _prep_call(token_ids, segment_table, positional_table)
  civ = type_token_ids * L + jnp.arange(L, dtype=jnp.int32)[None, :]
  tok4 = token_ids.reshape(NW, NCH, 2, CH)
  civ4 = civ.reshape(NW, NCH, 2, CH)
  outputs = _sc_embed(tok4, civ4, token_table, comb)
  attention_mask = mask.reshape(B, 1, 1, L)
  return outputs, attention_mask
